# baseline (speedup 1.0000x reference)
import jax
import jax.numpy as jnp
from jax import lax
from jax.experimental import pallas as pl
from jax.experimental.pallas import tpu as pltpu
from jax.experimental.pallas import tpu_sc as plsc
from jax._src.pallas import mpmd as _plmpmd

QN = -127.0
QP = 127.0
MSB = 128

ROWS, COLS = 4096, 4096
BLOCK_ROWS = 256

NUM_WORKERS = 32
LANES = 16
CAP = 256
NPAD = 2048
ROWS_PER_W = ROWS // NUM_WORKERS


def _dense_body(alpha_ref, w_ref, o_ref):
    a = alpha_ref[0]
    q = jnp.clip(w_ref[...] / a, QN, QP)
    o_ref[...] = jnp.round(q) * a


_dense_quantize = pl.pallas_call(
    _dense_body,
    grid=(ROWS // BLOCK_ROWS,),
    in_specs=[
        pl.BlockSpec(memory_space=pltpu.SMEM),
        pl.BlockSpec((BLOCK_ROWS, COLS), lambda i: (i, 0)),
    ],
    out_specs=pl.BlockSpec((BLOCK_ROWS, COLS), lambda i: (i, 0)),
    out_shape=jax.ShapeDtypeStruct((ROWS, COLS), jnp.float32),
)


ROUND_MAGIC = 12582912.0


def _flip_body(idx_hbm, meta_hbm, alpha_hbm, w_hbm, wq_in, out_hbm,
               idx_v, meta_v, alpha_v, wslots, oslots, sem):
    del wq_in
    cid = lax.axis_index("c")
    sid = lax.axis_index("s")
    wid = sid * 2 + cid
    pltpu.sync_copy(meta_hbm, meta_v.at[pl.ds(0, 3 * NUM_WORKERS)])
    pltpu.sync_copy(alpha_hbm, alpha_v)
    base8 = pl.multiple_of(meta_v[pl.ds(wid, LANES)][0], 8)
    off = meta_v[pl.ds(NUM_WORKERS + wid, LANES)][0]
    cnt = meta_v[pl.ds(2 * NUM_WORKERS + wid, LANES)][0]
    pltpu.sync_copy(idx_hbm.at[pl.ds(base8, CAP + 8)],
                    idx_v.at[pl.ds(0, CAP + 8)])
    a = alpha_v[...]

    def flip_at(j):
        return idx_v[pl.ds(off + j, LANES)][0]

    def coords(idx):
        r = idx >> 12
        c = idx & (COLS - 1)
        cg = pl.multiple_of((c // LANES) * LANES, LANES)
        return r, c, cg

    def fire(j, carry):
        r, _, cg = coords(flip_at(j))
        pltpu.async_copy(w_hbm.at[r, pl.ds(cg, LANES)], wslots.at[j], sem)
        return carry

    lax.fori_loop(0, cnt, fire, 0)

    def drain(j, carry):
        pltpu.make_async_copy(w_hbm.at[0, pl.ds(0, LANES)], wslots.at[0],
                              sem).wait()
        return carry

    lax.fori_loop(0, cnt, drain, 0)

    def proc(j, h):
        idx = flip_at(j)
        prev = idx_v[pl.ds(jnp.maximum(off + j - 1, 0), LANES)][0]
        nxt = idx_v[pl.ds(off + j + 1, LANES)][0]
        head = jnp.logical_or(j == 0, (prev >> 4) != (idx >> 4))
        last = jnp.logical_or(j == cnt - 1, (nxt >> 4) != (idx >> 4))
        h = jnp.where(head, j, h)
        r, c, cg = coords(idx)
        lane = c - cg
        wgrp = wslots[j]
        sel = jnp.minimum(jnp.maximum(wgrp / a, QN), QP)
        vvec = (sel.astype(jnp.int32) ^ MSB).astype(jnp.float32) * a
        dense = ((sel + ROUND_MAGIC) - ROUND_MAGIC) * a

        @pl.when(head)
        def _():
            oslots[h] = dense

        cur = oslots[h]
        mask = lax.iota(jnp.int32, LANES) == lane
        oslots[h] = jnp.where(mask, vvec, cur)

        @pl.when(last)
        def _():
            pltpu.async_copy(oslots.at[h], out_hbm.at[r, pl.ds(cg, LANES)],
                             sem)

        return h

    lax.fori_loop(0, cnt, proc, jnp.int32(0))

    def sdrain(j, carry):
        idx = flip_at(j)
        nxt = idx_v[pl.ds(off + j + 1, LANES)][0]
        last = jnp.logical_or(j == cnt - 1, (nxt >> 4) != (idx >> 4))

        @pl.when(last)
        def _():
            pltpu.make_async_copy(w_hbm.at[0, pl.ds(0, LANES)], oslots.at[0],
                                  sem).wait()

        return carry

    lax.fori_loop(0, cnt, sdrain, 0)


_flip_scatter = _plmpmd._mpmd_map(
    [(plsc.VectorSubcoreMesh(core_axis_name="c", subcore_axis_name="s"),
      _flip_body)],
    out_types=jax.ShapeDtypeStruct((ROWS, COLS), jnp.float32),
    input_output_aliases={4: 0},
    scratch_types=[
        pltpu.VMEM((CAP + 8 + LANES,), jnp.int32),
        pltpu.VMEM((3 * NUM_WORKERS + LANES,), jnp.int32),
        pltpu.VMEM((LANES,), jnp.float32),
        pltpu.VMEM((CAP, LANES), jnp.float32),
        pltpu.VMEM((CAP, LANES), jnp.float32),
        pltpu.SemaphoreType.DMA,
    ],
)


def kernel(weight, alpha, flip_idx):
    alpha_eff = jnp.maximum(alpha[0], 1e-4)
    wq = _dense_quantize(alpha_eff.reshape(1), weight)

    nf = flip_idx.shape[0]
    fi = jnp.sort(flip_idx)
    rows = fi >> 12
    band = jnp.arange(NUM_WORKERS, dtype=jnp.int32) * ROWS_PER_W
    bounds = jnp.sum(rows[None, :] < band[:, None], axis=1,
                     dtype=jnp.int32)
    endb = jnp.concatenate([bounds[1:], jnp.full((1,), nf, jnp.int32)])
    cnts = endb - bounds
    base8 = (bounds // 8) * 8
    off = bounds - base8
    meta = jnp.concatenate([base8, off, cnts])
    idx_pad = jnp.concatenate(
        [fi, jnp.broadcast_to(fi[-1:], (NPAD - nf,))])
    alpha_vec = jnp.full((LANES,), alpha_eff, jnp.float32)
    out = _flip_scatter(idx_pad, meta, alpha_vec, weight, wq)
    return out


# speedup vs baseline: 1.0176x; 1.0176x over previous
import jax
import jax.numpy as jnp
from jax import lax
from jax.experimental import pallas as pl
from jax.experimental.pallas import tpu as pltpu
from jax.experimental.pallas import tpu_sc as plsc
from jax._src.pallas import mpmd as _plmpmd

QN = -127.0
QP = 127.0
MSB = 128

ROWS, COLS = 4096, 4096
BLOCK_ROWS = 512

NUM_WORKERS = 32
LANES = 16
CAP = 256
NPAD = 2048
ROWS_PER_W = ROWS // NUM_WORKERS


def _dense_body(alpha_ref, w_ref, o_ref):
    a = alpha_ref[0]
    q = jnp.clip(w_ref[...] / a, QN, QP)
    o_ref[...] = jnp.round(q) * a


_dense_quantize = pl.pallas_call(
    _dense_body,
    grid=(ROWS // BLOCK_ROWS,),
    in_specs=[
        pl.BlockSpec(memory_space=pltpu.SMEM),
        pl.BlockSpec((BLOCK_ROWS, COLS), lambda i: (i, 0)),
    ],
    out_specs=pl.BlockSpec((BLOCK_ROWS, COLS), lambda i: (i, 0)),
    out_shape=jax.ShapeDtypeStruct((ROWS, COLS), jnp.float32),
)


ROUND_MAGIC = 12582912.0


def _flip_body(idx_hbm, meta_hbm, alpha_hbm, w_hbm, wq_in, out_hbm,
               idx_v, meta_v, alpha_v, wslots, oslots, sem):
    del wq_in
    cid = lax.axis_index("c")
    sid = lax.axis_index("s")
    wid = sid * 2 + cid
    pltpu.sync_copy(meta_hbm, meta_v.at[pl.ds(0, 3 * NUM_WORKERS)])
    pltpu.sync_copy(alpha_hbm, alpha_v)
    base8 = pl.multiple_of(meta_v[pl.ds(wid, LANES)][0], 8)
    off = meta_v[pl.ds(NUM_WORKERS + wid, LANES)][0]
    cnt = meta_v[pl.ds(2 * NUM_WORKERS + wid, LANES)][0]
    pltpu.sync_copy(idx_hbm.at[pl.ds(base8, CAP + 8)],
                    idx_v.at[pl.ds(0, CAP + 8)])
    a = alpha_v[...]

    def flip_at(j):
        return idx_v[pl.ds(off + j, LANES)][0]

    def coords(idx):
        r = idx >> 12
        c = idx & (COLS - 1)
        cg = pl.multiple_of((c // LANES) * LANES, LANES)
        return r, c, cg

    def fire(j, carry):
        r, _, cg = coords(flip_at(j))
        pltpu.async_copy(w_hbm.at[r, pl.ds(cg, LANES)], wslots.at[j], sem)
        return carry

    lax.fori_loop(0, cnt, fire, 0)

    def drain(j, carry):
        pltpu.make_async_copy(w_hbm.at[0, pl.ds(0, LANES)], wslots.at[0],
                              sem).wait()
        return carry

    lax.fori_loop(0, cnt, drain, 0)

    def proc(j, h):
        idx = flip_at(j)
        prev = idx_v[pl.ds(jnp.maximum(off + j - 1, 0), LANES)][0]
        nxt = idx_v[pl.ds(off + j + 1, LANES)][0]
        head = jnp.logical_or(j == 0, (prev >> 4) != (idx >> 4))
        last = jnp.logical_or(j == cnt - 1, (nxt >> 4) != (idx >> 4))
        h = jnp.where(head, j, h)
        r, c, cg = coords(idx)
        lane = c - cg
        wgrp = wslots[j]
        sel = jnp.minimum(jnp.maximum(wgrp / a, QN), QP)
        vvec = (sel.astype(jnp.int32) ^ MSB).astype(jnp.float32) * a
        dense = ((sel + ROUND_MAGIC) - ROUND_MAGIC) * a

        @pl.when(head)
        def _():
            oslots[h] = dense

        cur = oslots[h]
        mask = lax.iota(jnp.int32, LANES) == lane
        oslots[h] = jnp.where(mask, vvec, cur)

        @pl.when(last)
        def _():
            pltpu.async_copy(oslots.at[h], out_hbm.at[r, pl.ds(cg, LANES)],
                             sem)

        return h

    lax.fori_loop(0, cnt, proc, jnp.int32(0))

    def sdrain(j, carry):
        idx = flip_at(j)
        nxt = idx_v[pl.ds(off + j + 1, LANES)][0]
        last = jnp.logical_or(j == cnt - 1, (nxt >> 4) != (idx >> 4))

        @pl.when(last)
        def _():
            pltpu.make_async_copy(w_hbm.at[0, pl.ds(0, LANES)], oslots.at[0],
                                  sem).wait()

        return carry

    lax.fori_loop(0, cnt, sdrain, 0)


_flip_scatter = _plmpmd._mpmd_map(
    [(plsc.VectorSubcoreMesh(core_axis_name="c", subcore_axis_name="s"),
      _flip_body)],
    out_types=jax.ShapeDtypeStruct((ROWS, COLS), jnp.float32),
    input_output_aliases={4: 0},
    scratch_types=[
        pltpu.VMEM((CAP + 8 + LANES,), jnp.int32),
        pltpu.VMEM((3 * NUM_WORKERS + LANES,), jnp.int32),
        pltpu.VMEM((LANES,), jnp.float32),
        pltpu.VMEM((CAP, LANES), jnp.float32),
        pltpu.VMEM((CAP, LANES), jnp.float32),
        pltpu.SemaphoreType.DMA,
    ],
)


def kernel(weight, alpha, flip_idx):
    alpha_eff = jnp.maximum(alpha[0], 1e-4)
    wq = _dense_quantize(alpha_eff.reshape(1), weight)

    nf = flip_idx.shape[0]
    fi = jnp.sort(flip_idx)
    rows = fi >> 12
    band = jnp.arange(NUM_WORKERS, dtype=jnp.int32) * ROWS_PER_W
    bounds = jnp.sum(rows[None, :] < band[:, None], axis=1,
                     dtype=jnp.int32)
    endb = jnp.concatenate([bounds[1:], jnp.full((1,), nf, jnp.int32)])
    cnts = endb - bounds
    base8 = (bounds // 8) * 8
    off = bounds - base8
    meta = jnp.concatenate([base8, off, cnts])
    idx_pad = jnp.concatenate(
        [fi, jnp.broadcast_to(fi[-1:], (NPAD - nf,))])
    alpha_vec = jnp.full((LANES,), alpha_eff, jnp.float32)
    out = _flip_scatter(idx_pad, meta, alpha_vec, weight, wq)
    return out
